# Initial kernel scaffold; baseline (speedup 1.0000x reference)
import jax
import jax.numpy as jnp
from jax.experimental import pallas as pl


def kernel(user_embed, item_embed, edge_index, edge_vals):
    raise NotImplementedError("write your pallas kernel here")


# speedup vs baseline: 7.2990x; 7.2990x over previous
import functools

import jax
import jax.numpy as jnp
from jax import lax
from jax.experimental import pallas as pl
from jax.experimental.pallas import tpu as pltpu
from jax.experimental.pallas import tpu_sc as plsc

N_USERS = 50000
N_ITEMS = 50000
N_TOTAL = N_USERS + N_ITEMS
EMB_DIM = 32
HALF = 16
N_EDGES = 1600000
N_HOPS = 3

NS = 16
K = 8
CHUNK = K * 128
STAGES = 98
EDGES_PER_TILE = STAGES * CHUNK
E_PAD = NS * EDGES_PER_TILE
N_PAD = 100096
ROWS_PER_TILE = N_PAD // NS

_mesh = plsc.VectorSubcoreMesh(core_axis_name="c", subcore_axis_name="s")


@functools.partial(
    pl.kernel,
    mesh=_mesh,
    out_type=jax.ShapeDtypeStruct((2, N_PAD, HALF), jnp.float32),
    compiler_params=pltpu.CompilerParams(use_tc_tiling_on_sc=False),
    scratch_types=[
        pltpu.VMEM((K, 128), jnp.int32),
        pltpu.VMEM((K, 128), jnp.int32),
        pltpu.VMEM((K, 128), jnp.float32),
        pltpu.VMEM((128, HALF), jnp.float32),
        pltpu.SemaphoreType.DMA,
        pltpu.VMEM_SHARED((N_PAD, HALF), jnp.float32),
    ],
)
def _hop(tab_hbm, row_hbm, col_hbm, val_hbm, zeros_hbm, out_hbm,
         row_v, col_v, val_v, msg_v, sem, acc_sh):
    c = lax.axis_index("c")
    s = lax.axis_index("s")

    pltpu.sync_copy(zeros_hbm, acc_sh.at[pl.ds(s * ROWS_PER_TILE, ROWS_PER_TILE)])
    plsc.subcore_barrier()

    base128 = s * (STAGES * K)

    def stage_body(stage, _):
        st = base128 + stage * K
        pltpu.sync_copy(row_hbm.at[pl.ds(st, K)], row_v)
        pltpu.sync_copy(col_hbm.at[pl.ds(st, K)], col_v)
        pltpu.sync_copy(val_hbm.at[pl.ds(st, K)], val_v)
        for j in range(K):
            pltpu.async_copy(tab_hbm.at[c].at[col_v.at[j]], msg_v, sem).wait()

            def scale_group(g, _):
                vv = val_v[j, pl.ds(g * 16, 16)]
                base = g * 16
                for e in range(16):
                    msg_v[base + e, :] = msg_v[base + e, :] * vv[e]
                return 0

            lax.fori_loop(0, 8, scale_group, 0)
            pltpu.sync_copy(msg_v, acc_sh.at[row_v.at[j]], add=True)
        return 0

    lax.fori_loop(0, STAGES, stage_body, 0)

    plsc.subcore_barrier()
    sl = pl.ds(s * ROWS_PER_TILE, ROWS_PER_TILE)
    pltpu.sync_copy(acc_sh.at[sl], out_hbm.at[c, sl])


def kernel(user_embed, item_embed, edge_index, edge_vals):
    all_embed = jnp.concatenate([user_embed, item_embed], axis=0)
    all_embed = jnp.pad(all_embed, ((0, N_PAD - N_TOTAL), (0, 0)))
    tab = jnp.stack([all_embed[:, :HALF], all_embed[:, HALF:]])

    pad = E_PAD - N_EDGES
    row = jnp.concatenate([edge_index[0], jnp.zeros((pad,), edge_index.dtype)])
    col = jnp.concatenate([edge_index[1], jnp.zeros((pad,), edge_index.dtype)])
    val = jnp.concatenate([edge_vals, jnp.zeros((pad,), edge_vals.dtype)])
    row = row.reshape(-1, 128)
    col = col.reshape(-1, 128)
    val = val.reshape(-1, 128)
    zeros = jnp.zeros((ROWS_PER_TILE, HALF), jnp.float32)

    tabs = [tab]
    for _ in range(N_HOPS):
        tabs.append(_hop(tabs[-1], row, col, val, zeros))

    embs = jnp.stack(
        [jnp.concatenate([t[0, :N_TOTAL], t[1, :N_TOTAL]], axis=-1) for t in tabs],
        axis=1,
    )
    return embs[:N_USERS], embs[N_USERS:]


# speedup vs baseline: 13.2749x; 1.8187x over previous
import functools

import jax
import jax.numpy as jnp
from jax import lax
from jax.experimental import pallas as pl
from jax.experimental.pallas import tpu as pltpu
from jax.experimental.pallas import tpu_sc as plsc

N_USERS = 50000
N_ITEMS = 50000
N_TOTAL = N_USERS + N_ITEMS
EMB_DIM = 32
HALF = 16
N_EDGES = 1600000
N_HOPS = 3

NS = 16
K = 8
CHUNK = K * 128
STAGES = 98
EDGES_PER_TILE = STAGES * CHUNK
E_PAD = NS * EDGES_PER_TILE
N_PAD = 100096
ROWS_PER_TILE = N_PAD // NS

_mesh = plsc.VectorSubcoreMesh(core_axis_name="c", subcore_axis_name="s")


@functools.partial(
    pl.kernel,
    mesh=_mesh,
    out_type=jax.ShapeDtypeStruct((2, N_PAD, HALF), jnp.float32),
    compiler_params=pltpu.CompilerParams(use_tc_tiling_on_sc=False),
    scratch_types=[
        pltpu.VMEM((K, 128), jnp.int32),
        pltpu.VMEM((K, 128), jnp.int32),
        pltpu.VMEM((K, 128), jnp.float32),
        pltpu.VMEM((K, 128), jnp.int32),
        pltpu.VMEM((K, 128), jnp.int32),
        pltpu.VMEM((K, 128), jnp.float32),
        pltpu.VMEM((128, HALF), jnp.float32),
        pltpu.VMEM((128, HALF), jnp.float32),
        pltpu.SemaphoreType.DMA,
        pltpu.SemaphoreType.DMA,
        pltpu.SemaphoreType.DMA,
        pltpu.VMEM_SHARED((N_PAD, HALF), jnp.float32),
    ],
)
def _hop(tab_hbm, row_hbm, col_hbm, val_hbm, zeros_hbm, out_hbm,
         row_a, col_a, val_a, row_b, col_b, val_b, msg_a, msg_b,
         esem, gsem, ssem, acc_sh):
    c = lax.axis_index("c")
    s = lax.axis_index("s")
    tab = tab_hbm.at[c]

    pltpu.sync_copy(zeros_hbm, acc_sh.at[pl.ds(s * ROWS_PER_TILE, ROWS_PER_TILE)])
    plsc.subcore_barrier()

    base128 = s * (STAGES * K)

    def issue_edges(st, bufs):
        row_r, col_r, val_r = bufs
        pltpu.async_copy(row_hbm.at[pl.ds(st, K)], row_r, esem)
        pltpu.async_copy(col_hbm.at[pl.ds(st, K)], col_r, esem)
        pltpu.async_copy(val_hbm.at[pl.ds(st, K)], val_r, esem)

    def drain_edges(st, bufs):
        row_r, col_r, val_r = bufs
        pltpu.make_async_copy(row_hbm.at[pl.ds(st, K)], row_r, esem).wait()
        pltpu.make_async_copy(col_hbm.at[pl.ds(st, K)], col_r, esem).wait()
        pltpu.make_async_copy(val_hbm.at[pl.ds(st, K)], val_r, esem).wait()

    def scale(mb, val_r, j):
        def scale_group(g, _):
            vv = val_r[j, pl.ds(g * 16, 16)]
            base = g * 16
            for e in range(16):
                mb[base + e, :] = mb[base + e, :] * vv[e]
            return 0

        lax.fori_loop(0, 8, scale_group, 0)

    def stage_block(bufs):
        row_r, col_r, val_r = bufs
        mbs = (msg_a, msg_b)
        gather_h = [None, None]
        scatter_h = []
        gather_h[0] = pltpu.async_copy(tab.at[col_r.at[0]], mbs[0], gsem)
        for j in range(K):
            mb = mbs[j % 2]
            if j + 1 < K:
                if len(scatter_h) > 1:
                    scatter_h.pop(0).wait()
                gather_h[(j + 1) % 2] = pltpu.async_copy(
                    tab.at[col_r.at[j + 1]], mbs[(j + 1) % 2], gsem)
            gather_h[j % 2].wait()
            scale(mb, val_r, j)
            scatter_h.append(
                pltpu.async_copy(mb, acc_sh.at[row_r.at[j]], ssem, add=True))
        for h in scatter_h:
            h.wait()

    bufs_a = (row_a, col_a, val_a)
    bufs_b = (row_b, col_b, val_b)

    issue_edges(base128, bufs_a)

    def pair_body(t, _):
        st0 = base128 + (2 * t) * K
        st1 = st0 + K
        st2 = st1 + K
        drain_edges(st0, bufs_a)
        issue_edges(st1, bufs_b)
        stage_block(bufs_a)
        drain_edges(st1, bufs_b)

        @pl.when(t + 1 < STAGES // 2)
        def _():
            issue_edges(st2, bufs_a)

        stage_block(bufs_b)
        return 0

    lax.fori_loop(0, STAGES // 2, pair_body, 0)

    plsc.subcore_barrier()
    sl = pl.ds(s * ROWS_PER_TILE, ROWS_PER_TILE)
    pltpu.sync_copy(acc_sh.at[sl], out_hbm.at[c, sl])


def kernel(user_embed, item_embed, edge_index, edge_vals):
    all_embed = jnp.concatenate([user_embed, item_embed], axis=0)
    all_embed = jnp.pad(all_embed, ((0, N_PAD - N_TOTAL), (0, 0)))
    tab = jnp.stack([all_embed[:, :HALF], all_embed[:, HALF:]])

    pad = E_PAD - N_EDGES
    row = jnp.concatenate([edge_index[0], jnp.zeros((pad,), edge_index.dtype)])
    col = jnp.concatenate([edge_index[1], jnp.zeros((pad,), edge_index.dtype)])
    val = jnp.concatenate([edge_vals, jnp.zeros((pad,), edge_vals.dtype)])
    row = row.reshape(-1, 128)
    col = col.reshape(-1, 128)
    val = val.reshape(-1, 128)
    zeros = jnp.zeros((ROWS_PER_TILE, HALF), jnp.float32)

    tabs = [tab]
    for _ in range(N_HOPS):
        tabs.append(_hop(tabs[-1], row, col, val, zeros))

    embs = jnp.stack(
        [jnp.concatenate([t[0, :N_TOTAL], t[1, :N_TOTAL]], axis=-1) for t in tabs],
        axis=1,
    )
    return embs[:N_USERS], embs[N_USERS:]
